# Initial kernel scaffold; baseline (speedup 1.0000x reference)
import jax
import jax.numpy as jnp
from jax.experimental import pallas as pl


def kernel(x, W1, g1, b1, W2, g2, b2, W3, g3, b3):
    raise NotImplementedError("write your pallas kernel here")


# speedup vs baseline: 2.1783x; 2.1783x over previous
import jax
import jax.numpy as jnp
from jax import lax
from jax.experimental import pallas as pl

B, CIN, CB, COUT, H, W = 4, 128, 32, 128, 192, 192
HW = H * W
NCH = 4
NBLK = HW // NCH
NTOT = float(B * HW)
EPS = 1e-5
F32 = jnp.float32


def _scale_shift(s_ref, ss_ref, g_ref, b_ref):
    mean = s_ref[0] * (1.0 / NTOT)
    var = ss_ref[0] * (1.0 / NTOT) - mean * mean
    scale = g_ref[0] * lax.rsqrt(var + EPS)
    shift = b_ref[0] - mean * scale
    return scale, shift


def _p1_kernel(x_ref, w1_ref, y1_ref, s_ref, ss_ref):
    y = jnp.dot(w1_ref[...], x_ref[0], preferred_element_type=F32)
    y1_ref[0] = y
    first = (pl.program_id(0) == 0) & (pl.program_id(1) == 0)

    @pl.when(first)
    def _():
        s_ref[...] = jnp.zeros_like(s_ref)
        ss_ref[...] = jnp.zeros_like(ss_ref)

    s_ref[...] += jnp.sum(y, axis=1)[None]
    ss_ref[...] += jnp.sum(y * y, axis=1)[None]


def _p2_kernel(y1_ref, w2_ref, s1_ref, ss1_ref, g1_ref, b1_ref,
               y2_ref, s_ref, ss_ref):
    scale, shift = _scale_shift(s1_ref, ss1_ref, g1_ref, b1_ref)
    z1 = jnp.maximum(y1_ref[0] * scale[:, None] + shift[:, None], 0.0)

    wpos = lax.broadcasted_iota(jnp.int32, (1, HW), 1) % W
    mask_l = (wpos != 0).astype(F32)
    mask_r = (wpos != W - 1).astype(F32)

    acc = jnp.zeros((CB, HW), F32)
    for ky in range(3):
        for kx in range(3):
            sh = -(W * (ky - 1) + (kx - 1))
            if sh > 0:
                zs = jnp.concatenate(
                    [jnp.zeros((CB, sh), F32), z1[:, :HW - sh]], axis=1)
            elif sh < 0:
                zs = jnp.concatenate(
                    [z1[:, -sh:], jnp.zeros((CB, -sh), F32)], axis=1)
            else:
                zs = z1
            if kx == 0:
                zs = zs * mask_l
            elif kx == 2:
                zs = zs * mask_r
            acc += jnp.dot(w2_ref[ky, kx], zs, preferred_element_type=F32)

    y2_ref[0] = acc
    first = pl.program_id(0) == 0

    @pl.when(first)
    def _():
        s_ref[...] = jnp.zeros_like(s_ref)
        ss_ref[...] = jnp.zeros_like(ss_ref)

    s_ref[...] += jnp.sum(acc, axis=1)[None]
    ss_ref[...] += jnp.sum(acc * acc, axis=1)[None]


def _p3_kernel(y2_ref, w3_ref, s2_ref, ss2_ref, g2_ref, b2_ref,
               z2_ref, s_ref, ss_ref):
    scale, shift = _scale_shift(s2_ref, ss2_ref, g2_ref, b2_ref)
    z2 = jnp.maximum(y2_ref[0] * scale[:, None] + shift[:, None], 0.0)
    z2_ref[0] = z2
    y3 = jnp.dot(w3_ref[...], z2, preferred_element_type=F32)
    first = (pl.program_id(0) == 0) & (pl.program_id(1) == 0)

    @pl.when(first)
    def _():
        s_ref[...] = jnp.zeros_like(s_ref)
        ss_ref[...] = jnp.zeros_like(ss_ref)

    s_ref[...] += jnp.sum(y3, axis=1)[None]
    ss_ref[...] += jnp.sum(y3 * y3, axis=1)[None]


def _p4_kernel(z2_ref, x_ref, w3_ref, s3_ref, ss3_ref, g3_ref, b3_ref,
               out_ref):
    scale, shift = _scale_shift(s3_ref, ss3_ref, g3_ref, b3_ref)
    y3 = jnp.dot(w3_ref[...], z2_ref[0], preferred_element_type=F32)
    out_ref[0] = jnp.maximum(
        y3 * scale[:, None] + shift[:, None] + x_ref[0], 0.0)


def kernel(x, W1, g1, b1, W2, g2, b2, W3, g3, b3):
    xf = x.reshape(B, CIN, HW)
    w1 = W1.reshape(CB, CIN)
    w2 = jnp.transpose(W2, (2, 3, 0, 1))
    w3 = W3.reshape(COUT, CB)
    g1r, b1r = g1.reshape(1, CB), b1.reshape(1, CB)
    g2r, b2r = g2.reshape(1, CB), b2.reshape(1, CB)
    g3r, b3r = g3.reshape(1, COUT), b3.reshape(1, COUT)

    full = lambda shape: pl.BlockSpec(shape, lambda *_: (0,) * len(shape))

    y1, s1, ss1 = pl.pallas_call(
        _p1_kernel,
        grid=(B, NCH),
        in_specs=[
            pl.BlockSpec((1, CIN, NBLK), lambda b, n: (b, 0, n)),
            full((CB, CIN)),
        ],
        out_specs=[
            pl.BlockSpec((1, CB, NBLK), lambda b, n: (b, 0, n)),
            full((1, CB)),
            full((1, CB)),
        ],
        out_shape=[
            jax.ShapeDtypeStruct((B, CB, HW), F32),
            jax.ShapeDtypeStruct((1, CB), F32),
            jax.ShapeDtypeStruct((1, CB), F32),
        ],
    )(xf, w1)

    y2, s2, ss2 = pl.pallas_call(
        _p2_kernel,
        grid=(B,),
        in_specs=[
            pl.BlockSpec((1, CB, HW), lambda b: (b, 0, 0)),
            full((3, 3, CB, CB)),
            full((1, CB)), full((1, CB)), full((1, CB)), full((1, CB)),
        ],
        out_specs=[
            pl.BlockSpec((1, CB, HW), lambda b: (b, 0, 0)),
            full((1, CB)),
            full((1, CB)),
        ],
        out_shape=[
            jax.ShapeDtypeStruct((B, CB, HW), F32),
            jax.ShapeDtypeStruct((1, CB), F32),
            jax.ShapeDtypeStruct((1, CB), F32),
        ],
    )(y1, w2, s1, ss1, g1r, b1r)

    z2, s3, ss3 = pl.pallas_call(
        _p3_kernel,
        grid=(B, NCH),
        in_specs=[
            pl.BlockSpec((1, CB, NBLK), lambda b, n: (b, 0, n)),
            full((COUT, CB)),
            full((1, CB)), full((1, CB)), full((1, CB)), full((1, CB)),
        ],
        out_specs=[
            pl.BlockSpec((1, CB, NBLK), lambda b, n: (b, 0, n)),
            full((1, COUT)),
            full((1, COUT)),
        ],
        out_shape=[
            jax.ShapeDtypeStruct((B, CB, HW), F32),
            jax.ShapeDtypeStruct((1, COUT), F32),
            jax.ShapeDtypeStruct((1, COUT), F32),
        ],
    )(y2, w3, s2, ss2, g2r, b2r)

    out = pl.pallas_call(
        _p4_kernel,
        grid=(B, NCH),
        in_specs=[
            pl.BlockSpec((1, CB, NBLK), lambda b, n: (b, 0, n)),
            pl.BlockSpec((1, CIN, NBLK), lambda b, n: (b, 0, n)),
            full((COUT, CB)),
            full((1, COUT)), full((1, COUT)), full((1, COUT)), full((1, COUT)),
        ],
        out_specs=pl.BlockSpec((1, COUT, NBLK), lambda b, n: (b, 0, n)),
        out_shape=jax.ShapeDtypeStruct((B, COUT, HW), F32),
    )(z2, xf, w3, s3, ss3, g3r, b3r)

    return out.reshape(B, COUT, H, W)
